# Initial kernel scaffold; baseline (speedup 1.0000x reference)
import jax
import jax.numpy as jnp
from jax.experimental import pallas as pl


def kernel(query, grid, xyz_min, xyz_max):
    raise NotImplementedError("write your pallas kernel here")


# speedup vs baseline: 1.0074x; 1.0074x over previous
from functools import partial

import jax
import jax.numpy as jnp
from jax.experimental import pallas as pl
from jax.experimental.pallas import tpu as pltpu


def _dg_kernel(iota_ref, q_ref, grid_ref, out_ref, *, Nx, Ny, Nz, C):
    tm = q_ref.shape[1]

    u = q_ref[0:1, :]
    v = q_ref[1:2, :]
    w = q_ref[2:3, :]
    gi = iota_ref[0:Nx, :]
    gj = iota_ref[0:Ny, :]
    gk = iota_ref[0:Nz, :]

    hat_u = jnp.maximum(0.0, 1.0 - jnp.abs(u - gi))
    hat_v = jnp.maximum(0.0, 1.0 - jnp.abs(v - gj)).astype(jnp.bfloat16)
    hat_w = jnp.maximum(0.0, 1.0 - jnp.abs(w - gk)).astype(jnp.bfloat16)

    w_vw = (hat_v[:, None, :] * hat_w[None, :, :]).reshape(Ny * Nz, tm)

    b = jnp.dot(grid_ref[...], w_vw, preferred_element_type=jnp.float32)

    rows = []
    for c in range(C):
        bc = b[c * Nx:(c + 1) * Nx, :]
        rows.append(jnp.sum(bc * hat_u, axis=0, keepdims=True))
    out_ref[...] = jnp.concatenate(rows, axis=0)


def _dense_grid(query, grid, xyz_min, xyz_max, *, tm=512):
    _, C, Nx, Ny, Nz = grid.shape
    lead_shape = query.shape[:-1]

    q = query.reshape(-1, 3).astype(jnp.float32)
    M = q.shape[0]
    t = (q - xyz_min) / (xyz_max - xyz_min)
    sizes = jnp.array([Nx - 1, Ny - 1, Nz - 1], jnp.float32)
    q_idx = (t * sizes).T

    tm = max(128, (tm // 128) * 128)
    m_pad128 = pl.cdiv(M, 128) * 128
    tm_eff = min(tm, m_pad128)
    M_pad = pl.cdiv(M, tm_eff) * tm_eff
    q_soa = jnp.pad(q_idx, ((0, 0), (0, M_pad - M)))

    grid_mat = grid[0].reshape(C * Nx, Ny * Nz).astype(jnp.bfloat16)
    S = max(Nx, Ny, Nz)
    iota_col = jnp.arange(S, dtype=jnp.float32)[:, None]

    out_t = pl.pallas_call(
        partial(_dg_kernel, Nx=Nx, Ny=Ny, Nz=Nz, C=C),
        out_shape=jax.ShapeDtypeStruct((C, M_pad), jnp.float32),
        grid=(M_pad // tm_eff,),
        in_specs=[
            pl.BlockSpec((S, 1), lambda m: (0, 0)),
            pl.BlockSpec((3, tm_eff), lambda m: (0, m)),
            pl.BlockSpec((C * Nx, Ny * Nz), lambda m: (0, 0)),
        ],
        out_specs=pl.BlockSpec((C, tm_eff), lambda m: (0, m)),
        compiler_params=pltpu.CompilerParams(
            dimension_semantics=("parallel",),
            vmem_limit_bytes=32 * 1024 * 1024,
        ),
    )(iota_col, q_soa, grid_mat)

    return out_t[:, :M].T.reshape(*lead_shape, C)


def kernel(query, grid, xyz_min, xyz_max):
    return _dense_grid(query, grid, xyz_min, xyz_max)
